# Initial kernel scaffold; baseline (speedup 1.0000x reference)
import jax
import jax.numpy as jnp
from jax.experimental import pallas as pl


def kernel(x, edge_index, batch, W1g, b1g, W2g, b2g, W1t, b1t, W2t, b2t):
    raise NotImplementedError("write your pallas kernel here")


# speedup vs baseline: 27.8075x; 27.8075x over previous
import dataclasses
import functools

import jax
import jax.numpy as jnp
from jax import lax
from jax.experimental import pallas as pl
from jax.experimental.pallas import tpu as pltpu
from jax.experimental.pallas import tpu_sc as plsc

N = 10000
E = 320000
G = 64
CIN = 128
H = 128
OUT = 32
NC, NS = 2, 16
NW = NC * NS
EPT = E // NW
CH = 128
NFULL = EPT // CH
TAIL = EPT - NFULL * CH
NPAD = 10240
RPT = NPAD // NS
VSIZE = G * NPAD
VPT = VSIZE // NS
NB = 1024

_mesh = plsc.VectorSubcoreMesh(core_axis_name="core", subcore_axis_name="subcore")

_sc_params = pltpu.CompilerParams()
if "needs_layout_passes" in pltpu.CompilerParams.__dataclass_fields__:
    _sc_params = dataclasses.replace(_sc_params, needs_layout_passes=False)


@functools.partial(
    pl.kernel,
    out_type=jax.ShapeDtypeStruct((NC, NPAD), jnp.float32),
    mesh=_mesh,
    scratch_types=[
        pltpu.VMEM((CH,), jnp.int32),
        pltpu.VMEM((TAIL,), jnp.int32),
        pltpu.VMEM((CH,), jnp.float32),
        pltpu.VMEM((TAIL,), jnp.float32),
        pltpu.VMEM_SHARED((NPAD,), jnp.float32),
    ],
)
def _deg_call(dst_hbm, ones_hbm, z_hbm, deg_out, dstv, dstv_t, onesv, ones_t, deg_sh):
    c = lax.axis_index("core")
    s = lax.axis_index("subcore")
    w = c * NS + s
    pltpu.sync_copy(z_hbm, deg_sh.at[pl.ds(s * RPT, RPT)])
    pltpu.sync_copy(ones_hbm, onesv)
    pltpu.sync_copy(ones_hbm.at[pl.ds(0, TAIL)], ones_t)
    plsc.subcore_barrier()

    @pl.loop(0, NFULL)
    def _(i):
        base = pl.multiple_of(w * EPT + i * CH, 8)
        pltpu.sync_copy(dst_hbm.at[pl.ds(base, CH)], dstv)
        pltpu.sync_copy(onesv, deg_sh.at[dstv], add=True)

    base = pl.multiple_of(w * EPT + NFULL * CH, 8)
    pltpu.sync_copy(dst_hbm.at[pl.ds(base, TAIL)], dstv_t)
    pltpu.sync_copy(ones_t, deg_sh.at[dstv_t], add=True)
    plsc.subcore_barrier()
    pltpu.sync_copy(deg_sh.at[pl.ds(s * RPT, RPT)], deg_out.at[c, pl.ds(s * RPT, RPT)])


def _mm_body(x_ref, w_ref, d0_ref, d1_ref, h2_ref, dinv_ref):
    dinv = lax.rsqrt(d0_ref[...] + d1_ref[...] + 1.0)
    h = jnp.dot(x_ref[...], w_ref[...], preferred_element_type=jnp.float32)
    h2_ref[...] = dinv * h
    dinv_ref[...] = dinv


_mm_call = pl.pallas_call(
    _mm_body,
    grid=(NPAD // NB,),
    in_specs=[
        pl.BlockSpec((NB, CIN), lambda i: (i, 0)),
        pl.BlockSpec((CIN, H), lambda i: (0, 0)),
        pl.BlockSpec((NB, 1), lambda i: (i, 0)),
        pl.BlockSpec((NB, 1), lambda i: (i, 0)),
    ],
    out_specs=[
        pl.BlockSpec((NB, H), lambda i: (i, 0)),
        pl.BlockSpec((NB, 1), lambda i: (i, 0)),
    ],
    out_shape=[
        jax.ShapeDtypeStruct((NPAD, H), jnp.float32),
        jax.ShapeDtypeStruct((NPAD, 1), jnp.float32),
    ],
)


@functools.partial(
    pl.kernel,
    out_type=jax.ShapeDtypeStruct((NC, NPAD, H), jnp.float32),
    mesh=_mesh,
    scratch_types=[
        pltpu.VMEM((CH,), jnp.int32),
        pltpu.VMEM((CH,), jnp.int32),
        pltpu.VMEM((TAIL,), jnp.int32),
        pltpu.VMEM((TAIL,), jnp.int32),
        pltpu.VMEM((CH, H), jnp.float32),
        pltpu.VMEM((TAIL, H), jnp.float32),
        pltpu.VMEM_SHARED((NPAD, H), jnp.float32),
    ],
    compiler_params=_sc_params,
)
def _agg_call(src_hbm, dst_hbm, h2_hbm, z2_hbm, agg_out,
              srcv, dstv, srcv_t, dstv_t, rows, rows_t, agg_sh):
    c = lax.axis_index("core")
    s = lax.axis_index("subcore")
    w = c * NS + s
    pltpu.sync_copy(z2_hbm, agg_sh.at[pl.ds(s * RPT, RPT)])
    plsc.subcore_barrier()

    def do_chunk(base, n, sv, dv, rw):
        pltpu.sync_copy(src_hbm.at[pl.ds(base, n)], sv)
        pltpu.sync_copy(dst_hbm.at[pl.ds(base, n)], dv)
        pltpu.sync_copy(h2_hbm.at[sv], rw)
        pltpu.sync_copy(rw, agg_sh.at[dv], add=True)

    @pl.loop(0, NFULL)
    def _(i):
        do_chunk(pl.multiple_of(w * EPT + i * CH, 8), CH, srcv, dstv, rows)

    do_chunk(pl.multiple_of(w * EPT + NFULL * CH, 8), TAIL, srcv_t, dstv_t,
             rows_t)
    plsc.subcore_barrier()
    pltpu.sync_copy(agg_sh.at[pl.ds(s * RPT, RPT)],
                    agg_out.at[c, pl.ds(s * RPT, RPT)])


@functools.partial(
    pl.kernel,
    out_type=jax.ShapeDtypeStruct((NC, VSIZE), jnp.float32),
    mesh=_mesh,
    scratch_types=[
        pltpu.VMEM((CH,), jnp.int32),
        pltpu.VMEM((CH,), jnp.int32),
        pltpu.VMEM((TAIL,), jnp.int32),
        pltpu.VMEM((TAIL,), jnp.int32),
        pltpu.VMEM((CH,), jnp.float32),
        pltpu.VMEM((CH,), jnp.int32),
        pltpu.VMEM((TAIL,), jnp.float32),
        pltpu.VMEM((TAIL,), jnp.int32),
        pltpu.VMEM((NPAD,), jnp.float32),
        pltpu.VMEM((NPAD,), jnp.int32),
        pltpu.VMEM_SHARED((VSIZE,), jnp.float32),
    ],
    compiler_params=_sc_params,
)
def _vtab_call(src_hbm, dst_hbm, dinv_hbm, batch_hbm, z1_hbm, v_out,
               srcv, dstv, srcv_t, dstv_t, vvals, vidx, vvals_t, vidx_t,
               dinvv, batchv, v_sh):
    c = lax.axis_index("core")
    s = lax.axis_index("subcore")
    w = c * NS + s
    pltpu.sync_copy(z1_hbm, v_sh.at[pl.ds(s * VPT, VPT)])
    pltpu.sync_copy(dinv_hbm, dinvv)
    pltpu.sync_copy(batch_hbm, batchv)
    plsc.subcore_barrier()

    def do_chunk(base, n, sv, dv, vv, vi):
        pltpu.sync_copy(src_hbm.at[pl.ds(base, n)], sv)
        pltpu.sync_copy(dst_hbm.at[pl.ds(base, n)], dv)

        @pl.loop(0, n // 16)
        def _(j):
            s16 = sv[pl.ds(j * 16, 16)]
            d16 = dv[pl.ds(j * 16, 16)]
            vv[pl.ds(j * 16, 16)] = plsc.load_gather(dinvv, [d16])
            vi[pl.ds(j * 16, 16)] = plsc.load_gather(batchv, [d16]) * NPAD + s16

        pltpu.sync_copy(vv, v_sh.at[vi], add=True)

    @pl.loop(0, NFULL)
    def _(i):
        do_chunk(pl.multiple_of(w * EPT + i * CH, 8), CH, srcv, dstv, vvals,
                 vidx)

    do_chunk(pl.multiple_of(w * EPT + NFULL * CH, 8), TAIL, srcv_t, dstv_t,
             vvals_t, vidx_t)
    plsc.subcore_barrier()
    pltpu.sync_copy(v_sh.at[pl.ds(s * VPT, VPT)],
                    v_out.at[c, pl.ds(s * VPT, VPT)])


def _final_body(a0_ref, a1_ref, h2_ref, dcol_ref, drow_ref, brow_ref, v0_ref,
                v1_ref, b1_ref, w2g_ref, w2t_ref, b2g_ref, b2t_ref, out_ref,
                u_acc, cnt_acc):
    i = pl.program_id(0)

    @pl.when(i == 0)
    def _():
        u_acc[...] = jnp.zeros_like(u_acc)
        cnt_acc[...] = jnp.zeros_like(cnt_acc)

    dinv = dcol_ref[...]
    a = a0_ref[...] + a1_ref[...] + h2_ref[...]
    h1 = dinv * a + b1_ref[...]
    rd = dinv * jnp.maximum(h1, 0.0)
    gids = lax.broadcasted_iota(jnp.int32, (G, NB), 0)
    onehot = brow_ref[...] == gids
    v_eff = v0_ref[...] + v1_ref[...] + jnp.where(onehot, drow_ref[...], 0.0)
    u_acc[...] += jnp.dot(v_eff, rd, preferred_element_type=jnp.float32)
    cnt_acc[...] += jnp.sum(onehot.astype(jnp.float32), axis=1, keepdims=True)

    @pl.when(i == pl.num_programs(0) - 1)
    def _():
        cnt = cnt_acc[...]
        us = u_acc[...] / jnp.maximum(cnt, 1.0)
        nz = jnp.where(cnt > 0, 1.0, 0.0)
        pg = jnp.dot(us[:, :64], w2g_ref[...],
                     preferred_element_type=jnp.float32) + b2g_ref[...] * nz
        pt = jnp.dot(us[:, 64:], w2t_ref[...],
                     preferred_element_type=jnp.float32) + b2t_ref[...] * nz
        diff = pt - pg + 1e-6
        dist = jnp.sqrt(jnp.sum(diff * diff, axis=1, keepdims=True))
        out_ref[...] = jnp.sum(dist).reshape(1, 1) / G


_final_call = pl.pallas_call(
    _final_body,
    grid=(NPAD // NB,),
    in_specs=[
        pl.BlockSpec((NB, H), lambda i: (i, 0)),
        pl.BlockSpec((NB, H), lambda i: (i, 0)),
        pl.BlockSpec((NB, H), lambda i: (i, 0)),
        pl.BlockSpec((NB, 1), lambda i: (i, 0)),
        pl.BlockSpec((1, NB), lambda i: (0, i)),
        pl.BlockSpec((1, NB), lambda i: (0, i)),
        pl.BlockSpec((G, NB), lambda i: (0, i)),
        pl.BlockSpec((G, NB), lambda i: (0, i)),
        pl.BlockSpec((1, H), lambda i: (0, 0)),
        pl.BlockSpec((64, OUT), lambda i: (0, 0)),
        pl.BlockSpec((64, OUT), lambda i: (0, 0)),
        pl.BlockSpec((1, OUT), lambda i: (0, 0)),
        pl.BlockSpec((1, OUT), lambda i: (0, 0)),
    ],
    out_specs=pl.BlockSpec((1, 1), lambda i: (0, 0)),
    out_shape=jax.ShapeDtypeStruct((1, 1), jnp.float32),
    scratch_shapes=[
        pltpu.VMEM((G, H), jnp.float32),
        pltpu.VMEM((G, 1), jnp.float32),
    ],
)


def kernel(x, edge_index, batch, W1g, b1g, W2g, b2g, W1t, b1t, W2t, b2t):
    src = edge_index[0]
    dst = edge_index[1]
    Wcat = jnp.concatenate([W1g, W1t], axis=1)
    b1cat = jnp.concatenate([b1g, b1t]).reshape(1, H)
    ones_ch = jnp.ones((CH,), jnp.float32)
    z1 = jnp.zeros((RPT,), jnp.float32)
    z2 = jnp.zeros((RPT, H), jnp.float32)
    zv = jnp.zeros((VPT,), jnp.float32)

    x_pad = jnp.pad(x, ((0, NPAD - N), (0, 0)))
    batch_pad = jnp.pad(batch, (0, NPAD - N), constant_values=G)

    deg_parts = _deg_call(dst, ones_ch, z1)
    d0 = deg_parts[0].reshape(NPAD, 1)
    d1 = deg_parts[1].reshape(NPAD, 1)
    h2, dinv = _mm_call(x_pad, Wcat, d0, d1)
    aggp = _agg_call(src, dst, h2, z2)
    vp = _vtab_call(src, dst, dinv.reshape(NPAD), batch_pad, zv)
    out = _final_call(
        aggp[0], aggp[1], h2, dinv, dinv.reshape(1, NPAD),
        batch_pad.reshape(1, NPAD), vp[0].reshape(G, NPAD),
        vp[1].reshape(G, NPAD), b1cat, W2g, W2t,
        b2g.reshape(1, OUT), b2t.reshape(1, OUT))
    return out.reshape(())
